# baseline (speedup 1.0000x reference)
import functools

import jax
import jax.numpy as jnp
from jax import lax
from jax.experimental import pallas as pl
from jax.experimental.pallas import tpu as pltpu
from jax.experimental.pallas import tpu_sc as plsc

_BATCH = 16384
_EMBED_DIM = 64
_NUM_IDX = 2 * _BATCH
_BN_EPS = 1e-3

_NC = 2
_NS = 16
_NW = _NC * _NS
_IDX_PER_W = _NUM_IDX // _NW
_CHUNK = 128
_NCHUNK = _IDX_PER_W // _CHUNK


def _sc_gather_body(table_hbm, idx_hbm, out_hbm, *rest):
    idx_bufs = rest[:_NCHUNK]
    rows_v, sem = rest[_NCHUNK], rest[_NCHUNK + 1]
    wid = lax.axis_index("s") * _NC + lax.axis_index("c")
    row0 = wid * _NCHUNK
    for j in range(_NCHUNK):
        pltpu.sync_copy(idx_hbm.at[row0 + j], idx_bufs[j])
    copies = []
    for j in range(_NCHUNK):
        c = pltpu.async_copy(
            table_hbm.at[idx_bufs[j]],
            rows_v.at[pl.ds(j * _CHUNK, _CHUNK)],
            sem,
        )
        copies.append(c)
    for c in copies:
        c.wait()
    pltpu.sync_copy(rows_v, out_hbm.at[pl.ds(wid * _IDX_PER_W, _IDX_PER_W)])


@functools.lru_cache(maxsize=1)
def _sc_gather():
    return pl.kernel(
        _sc_gather_body,
        out_type=jax.ShapeDtypeStruct((_NUM_IDX, _EMBED_DIM), jnp.float32),
        mesh=plsc.VectorSubcoreMesh(core_axis_name="c", subcore_axis_name="s",
                                    num_cores=_NC, num_subcores=_NS),
        scratch_types=(
            [pltpu.VMEM((_CHUNK,), jnp.int32) for _ in range(_NCHUNK)]
            + [
                pltpu.VMEM((_IDX_PER_W, _EMBED_DIM), jnp.float32),
                pltpu.SemaphoreType.DMA,
            ]
        ),
        compiler_params=pltpu.CompilerParams(use_tc_tiling_on_sc=False),
    )


def _tc_body(h0_ref, w1_ref, c1_ref, w2_ref, c2_ref, wf_ref, cf_ref, out_ref):
    h0 = h0_ref[...]
    h1 = jnp.dot(h0, w1_ref[...], preferred_element_type=jnp.float32,
                 precision=lax.Precision.HIGHEST)
    h1 = jnp.maximum(h1 + c1_ref[...], 0.0)
    h2 = jnp.dot(h1, w2_ref[...], preferred_element_type=jnp.float32,
                 precision=lax.Precision.HIGHEST)
    h2 = jnp.maximum(h2 + c2_ref[...], 0.0)
    gmf = h0[:, :_EMBED_DIM] * h0[:, _EMBED_DIM:]
    cat = jnp.concatenate([gmf, h2], axis=1)
    out = jnp.dot(cat, wf_ref[...], preferred_element_type=jnp.float32,
                  precision=lax.Precision.HIGHEST)
    out_ref[...] = out + cf_ref[...]


def _tc_mlp(h0, w1, c1, w2, c2, wf, cf, block_b=2048):
    nblk = _BATCH // block_b
    rep2 = lambda i: (0, 0)
    return pl.pallas_call(
        _tc_body,
        grid=(nblk,),
        in_specs=[
            pl.BlockSpec((block_b, 2 * _EMBED_DIM), lambda i: (i, 0)),
            pl.BlockSpec(w1.shape, rep2),
            pl.BlockSpec(c1.shape, rep2),
            pl.BlockSpec(w2.shape, rep2),
            pl.BlockSpec(c2.shape, rep2),
            pl.BlockSpec(wf.shape, rep2),
            pl.BlockSpec(cf.shape, rep2),
        ],
        out_specs=pl.BlockSpec((block_b, 1), lambda i: (i, 0)),
        out_shape=jax.ShapeDtypeStruct((_BATCH, 1), jnp.float32),
    )(h0, w1, c1, w2, c2, wf, cf)


def kernel(x, table, W1, b1, g1, be1, W2, b2, g2, be2, Wf, bf):
    s = jax.lax.rsqrt(jnp.float32(1.0 + _BN_EPS))
    w1 = W1 * (s * g1)[None, :]
    c1 = (b1 * s * g1 + be1)[None, :]
    w2 = W2 * (s * g2)[None, :]
    c2 = (b2 * s * g2 + be2)[None, :]
    cf = bf[None, :]

    idx = x.reshape(_NUM_IDX // _CHUNK, _CHUNK)
    emb = _sc_gather()(table, idx)
    h0 = emb.reshape(_BATCH, 2 * _EMBED_DIM)
    return _tc_mlp(h0, w1, c1, w2, c2, Wf, cf)


# speedup vs baseline: 7.2059x; 7.2059x over previous
import functools

import jax
import jax.numpy as jnp
from jax import lax
from jax.experimental import pallas as pl
from jax.experimental.pallas import tpu as pltpu
from jax.experimental.pallas import tpu_sc as plsc

_BATCH = 16384
_EMBED_DIM = 64
_NFEAT = 2 * _EMBED_DIM
_VOCAB = 2000000
_FLAT = _VOCAB * _EMBED_DIM
_BN_EPS = 1e-3

_NC = 2
_NS = 16
_NW = _NC * _NS
_ROW_PER_W = _NFEAT // _NW
_CHUNK = 128
_NCHUNK = _BATCH // _CHUNK
_FIRE = 16


def _sc_gather_body(flat_hbm, offs_hbm, out_hbm, idx_buf, rows_v, sem):
    wid = lax.axis_index("s") * _NC + lax.axis_index("c")
    for rr in range(_ROW_PER_W):
        r = wid * _ROW_PER_W + rr
        pltpu.sync_copy(offs_hbm.at[r], idx_buf)

        def grp(g, rr=rr):
            copies = []
            for k in range(_FIRE):
                q = g * _FIRE + k
                copies.append(pltpu.async_copy(
                    flat_hbm.at[idx_buf.at[q]],
                    rows_v.at[pl.ds(q * _CHUNK, _CHUNK)],
                    sem,
                ))
            for c in copies:
                c.wait()

        pl.loop(0, _NCHUNK // _FIRE)(grp)
        pltpu.sync_copy(rows_v, out_hbm.at[r])


@functools.lru_cache(maxsize=1)
def _sc_gather():
    return pl.kernel(
        _sc_gather_body,
        out_type=jax.ShapeDtypeStruct((_NFEAT, _BATCH), jnp.float32),
        mesh=plsc.VectorSubcoreMesh(core_axis_name="c", subcore_axis_name="s",
                                    num_cores=_NC, num_subcores=_NS),
        scratch_types=(
            pltpu.VMEM((_NCHUNK, _CHUNK), jnp.int32),
            pltpu.VMEM((_BATCH,), jnp.float32),
            pltpu.SemaphoreType.DMA,
        ),
        compiler_params=pltpu.CompilerParams(use_tc_tiling_on_sc=False),
    )


def _tc_body(h0_ref, w1_ref, c1_ref, w2_ref, c2_ref, wf_ref, cf_ref, out_ref):
    h0 = h0_ref[...]
    h1 = jnp.dot(w1_ref[...], h0, preferred_element_type=jnp.float32,
                 precision=lax.Precision.HIGHEST)
    h1 = jnp.maximum(h1 + c1_ref[...], 0.0)
    h2 = jnp.dot(w2_ref[...], h1, preferred_element_type=jnp.float32,
                 precision=lax.Precision.HIGHEST)
    h2 = jnp.maximum(h2 + c2_ref[...], 0.0)
    gmf = h0[:_EMBED_DIM] * h0[_EMBED_DIM:]
    cat = jnp.concatenate([gmf, h2], axis=0)
    out = jnp.dot(wf_ref[...], cat, preferred_element_type=jnp.float32,
                  precision=lax.Precision.HIGHEST)
    out_ref[...] = out + cf_ref[...]


def _tc_mlp(h0t, w1t, c1t, w2t, c2t, wft, cft, block_b=2048):
    nblk = _BATCH // block_b
    rep2 = lambda i: (0, 0)
    return pl.pallas_call(
        _tc_body,
        grid=(nblk,),
        in_specs=[
            pl.BlockSpec((_NFEAT, block_b), lambda i: (0, i)),
            pl.BlockSpec(w1t.shape, rep2),
            pl.BlockSpec(c1t.shape, rep2),
            pl.BlockSpec(w2t.shape, rep2),
            pl.BlockSpec(c2t.shape, rep2),
            pl.BlockSpec(wft.shape, rep2),
            pl.BlockSpec(cft.shape, rep2),
        ],
        out_specs=pl.BlockSpec((1, block_b), lambda i: (0, i)),
        out_shape=jax.ShapeDtypeStruct((1, _BATCH), jnp.float32),
    )(h0t, w1t, c1t, w2t, c2t, wft, cft)


def kernel(x, table, W1, b1, g1, be1, W2, b2, g2, be2, Wf, bf):
    s = jax.lax.rsqrt(jnp.float32(1.0 + _BN_EPS))
    w1t = (W1 * (s * g1)[None, :]).T
    c1t = (b1 * s * g1 + be1)[:, None]
    w2t = (W2 * (s * g2)[None, :]).T
    c2t = (b2 * s * g2 + be2)[:, None]
    wft = Wf.T
    cft = bf[:, None]

    flat = table.T.reshape(8, 8, 15625, 128).transpose(0, 2, 1, 3).reshape(_FLAT)

    j = jnp.arange(_EMBED_DIM, dtype=jnp.int32)
    cj = (j >> 3) * jnp.int32(16000000) + (j & 7) * jnp.int32(128)
    p = ((x >> 7) << 10) + (x & 127)
    offs = (p.T[:, None, :] + cj[None, :, None]).reshape(_NFEAT, _NCHUNK, _CHUNK)

    h0t = _sc_gather()(flat, offs)
    out = _tc_mlp(h0t, w1t, c1t, w2t, c2t, wft, cft)
    return out.T


# speedup vs baseline: 8.2971x; 1.1514x over previous
import functools

import jax
import jax.numpy as jnp
from jax import lax
from jax.experimental import pallas as pl
from jax.experimental.pallas import tpu as pltpu
from jax.experimental.pallas import tpu_sc as plsc

_BATCH = 16384
_EMBED_DIM = 64
_NFEAT = 2 * _EMBED_DIM
_VOCAB = 2000000
_FLAT = _VOCAB * _EMBED_DIM
_ROWLEN = 16000000
_BN_EPS = 1e-3

_NC = 2
_NS = 16
_NW = _NC * _NS
_ROW_PER_W = _NFEAT // _NW
_CHUNK = 128
_NCHUNK = _BATCH // _CHUNK
_GRP = 16
_NGRP = _NCHUNK // _GRP


def _sc_gather_body(flat_hbm, p_hbm, out_hbm, idx_buf, row0_v, row1_v,
                    gsem, osem):
    wid = lax.axis_index("s") * _NC + lax.axis_index("c")
    l = wid // 16
    g = (wid % 16) // 2
    m0 = (wid % 2) * 4
    pltpu.sync_copy(p_hbm.at[l], idx_buf)

    bufs = (row0_v, row1_v)
    for rr in range(_ROW_PER_W):
        r = wid * _ROW_PER_W + rr
        cj = g * _ROWLEN + (m0 + rr) * _CHUNK
        view = flat_hbm.at[pl.ds(cj, _ROWLEN)]
        buf = bufs[rr % 2]

        if rr >= 2:
            pltpu.make_async_copy(buf, out_hbm.at[r - 2], osem).wait()

        for k in range(_GRP):
            pltpu.async_copy(view.at[idx_buf.at[k]],
                             buf.at[pl.ds(k * _CHUNK, _CHUNK)], gsem)

        def grp(gi, view=view, buf=buf):
            for k in range(_GRP):
                q = (gi + 1) * _GRP + k
                pltpu.async_copy(view.at[idx_buf.at[q]],
                                 buf.at[pl.ds(q * _CHUNK, _CHUNK)], gsem)
            pltpu.make_async_copy(
                flat_hbm.at[pl.ds(0, _GRP * _CHUNK)],
                buf.at[pl.ds(gi * _GRP * _CHUNK, _GRP * _CHUNK)],
                gsem).wait()

        pl.loop(0, _NGRP - 1)(grp)
        pltpu.make_async_copy(
            flat_hbm.at[pl.ds(0, _GRP * _CHUNK)],
            buf.at[pl.ds((_NGRP - 1) * _GRP * _CHUNK, _GRP * _CHUNK)],
            gsem).wait()
        pltpu.async_copy(buf, out_hbm.at[r], osem)

    for rr in range(2):
        r = wid * _ROW_PER_W + 2 + rr
        pltpu.make_async_copy(bufs[rr], out_hbm.at[r], osem).wait()


@functools.lru_cache(maxsize=1)
def _sc_gather():
    return pl.kernel(
        _sc_gather_body,
        out_type=jax.ShapeDtypeStruct((_NFEAT, _BATCH), jnp.float32),
        mesh=plsc.VectorSubcoreMesh(core_axis_name="c", subcore_axis_name="s",
                                    num_cores=_NC, num_subcores=_NS),
        scratch_types=(
            pltpu.VMEM((_NCHUNK, _CHUNK), jnp.int32),
            pltpu.VMEM((_BATCH,), jnp.float32),
            pltpu.VMEM((_BATCH,), jnp.float32),
            pltpu.SemaphoreType.DMA,
            pltpu.SemaphoreType.DMA,
        ),
        compiler_params=pltpu.CompilerParams(use_tc_tiling_on_sc=False),
    )


def _tc_body(h0_ref, w1_ref, c1_ref, w2_ref, c2_ref, wf_ref, cf_ref, out_ref):
    h0 = h0_ref[...]
    h1 = jnp.dot(w1_ref[...], h0, preferred_element_type=jnp.float32,
                 precision=lax.Precision.HIGHEST)
    h1 = jnp.maximum(h1 + c1_ref[...], 0.0)
    h2 = jnp.dot(w2_ref[...], h1, preferred_element_type=jnp.float32,
                 precision=lax.Precision.HIGHEST)
    h2 = jnp.maximum(h2 + c2_ref[...], 0.0)
    gmf = h0[:_EMBED_DIM] * h0[_EMBED_DIM:]
    cat = jnp.concatenate([gmf, h2], axis=0)
    out = jnp.dot(wf_ref[...], cat, preferred_element_type=jnp.float32,
                  precision=lax.Precision.HIGHEST)
    out_ref[...] = out + cf_ref[...]


def _tc_mlp(h0t, w1t, c1t, w2t, c2t, wft, cft, block_b=2048):
    nblk = _BATCH // block_b
    rep2 = lambda i: (0, 0)
    return pl.pallas_call(
        _tc_body,
        grid=(nblk,),
        in_specs=[
            pl.BlockSpec((_NFEAT, block_b), lambda i: (0, i)),
            pl.BlockSpec(w1t.shape, rep2),
            pl.BlockSpec(c1t.shape, rep2),
            pl.BlockSpec(w2t.shape, rep2),
            pl.BlockSpec(c2t.shape, rep2),
            pl.BlockSpec(wft.shape, rep2),
            pl.BlockSpec(cft.shape, rep2),
        ],
        out_specs=pl.BlockSpec((1, block_b), lambda i: (0, i)),
        out_shape=jax.ShapeDtypeStruct((1, _BATCH), jnp.float32),
    )(h0t, w1t, c1t, w2t, c2t, wft, cft)


def kernel(x, table, W1, b1, g1, be1, W2, b2, g2, be2, Wf, bf):
    s = jax.lax.rsqrt(jnp.float32(1.0 + _BN_EPS))
    w1t = (W1 * (s * g1)[None, :]).T
    c1t = (b1 * s * g1 + be1)[:, None]
    w2t = (W2 * (s * g2)[None, :]).T
    c2t = (b2 * s * g2 + be2)[:, None]
    wft = Wf.T
    cft = bf[:, None]

    flat = table.T.reshape(8, 8, 15625, 128).transpose(0, 2, 1, 3).reshape(_FLAT)

    p = ((x >> 7) << 10) + (x & 127)
    pr = p.T.reshape(2, _NCHUNK, _CHUNK)

    h0t = _sc_gather()(flat, pr)
    out = _tc_mlp(h0t, w1t, c1t, w2t, c2t, wft, cft)
    return out.T


# speedup vs baseline: 8.3303x; 1.0040x over previous
import functools

import jax
import jax.numpy as jnp
from jax import lax
from jax.experimental import pallas as pl
from jax.experimental.pallas import tpu as pltpu
from jax.experimental.pallas import tpu_sc as plsc

_BATCH = 16384
_EMBED_DIM = 64
_NFEAT = 2 * _EMBED_DIM
_VOCAB = 2000000
_FLAT = _VOCAB * _EMBED_DIM
_ROWLEN = 16000000
_BN_EPS = 1e-3

_NC = 2
_NS = 16
_NW = _NC * _NS
_ROW_PER_W = _NFEAT // _NW
_CHUNK = 128
_NCHUNK = _BATCH // _CHUNK
_GRP = 16
_NGRP = _NCHUNK // _GRP


def _sc_gather_body(flat_hbm, p_hbm, out_hbm, idx_buf, row0_v, row1_v,
                    gsem0, gsem1, osem):
    wid = lax.axis_index("s") * _NC + lax.axis_index("c")
    l = wid // 16
    g = (wid % 16) // 2
    m0 = (wid % 2) * 4
    pltpu.sync_copy(p_hbm.at[l], idx_buf)

    bufs = (row0_v, row1_v)
    sems = (gsem0, gsem1)
    views = [flat_hbm.at[pl.ds(g * _ROWLEN + (m0 + rr) * _CHUNK, _ROWLEN)]
             for rr in range(_ROW_PER_W)]

    def issue_grp(view, buf, sem, gi):
        for k in range(_GRP):
            q = gi * _GRP + k
            pltpu.async_copy(view.at[idx_buf.at[q]],
                             buf.at[pl.ds(q * _CHUNK, _CHUNK)], sem)

    def drain_grp(buf, sem):
        pltpu.make_async_copy(flat_hbm.at[pl.ds(0, _GRP * _CHUNK)],
                              buf.at[pl.ds(0, _GRP * _CHUNK)], sem).wait()

    def row_prime(rr):
        issue_grp(views[rr], bufs[rr % 2], sems[rr % 2], 0)

    def row_main(rr):
        view, buf, sem = views[rr], bufs[rr % 2], sems[rr % 2]

        def body(gi, view=view, buf=buf, sem=sem):
            issue_grp(view, buf, sem, gi + 1)
            drain_grp(buf, sem)

        pl.loop(0, _NGRP - 1)(body)

    def row_finish(rr):
        buf, sem = bufs[rr % 2], sems[rr % 2]
        drain_grp(buf, sem)
        pltpu.async_copy(buf, out_hbm.at[wid * _ROW_PER_W + rr], osem)

    def out_wait(rr):
        pltpu.make_async_copy(bufs[rr % 2],
                              out_hbm.at[wid * _ROW_PER_W + rr], osem).wait()

    row_prime(0)
    row_main(0)
    row_prime(1)
    row_finish(0)
    row_main(1)
    out_wait(0)
    row_prime(2)
    row_finish(1)
    row_main(2)
    out_wait(1)
    row_prime(3)
    row_finish(2)
    row_main(3)
    row_finish(3)
    out_wait(2)
    out_wait(3)


@functools.lru_cache(maxsize=1)
def _sc_gather():
    return pl.kernel(
        _sc_gather_body,
        out_type=jax.ShapeDtypeStruct((_NFEAT, _BATCH), jnp.float32),
        mesh=plsc.VectorSubcoreMesh(core_axis_name="c", subcore_axis_name="s",
                                    num_cores=_NC, num_subcores=_NS),
        scratch_types=(
            pltpu.VMEM((_NCHUNK, _CHUNK), jnp.int32),
            pltpu.VMEM((_BATCH,), jnp.float32),
            pltpu.VMEM((_BATCH,), jnp.float32),
            pltpu.SemaphoreType.DMA,
            pltpu.SemaphoreType.DMA,
            pltpu.SemaphoreType.DMA,
        ),
        compiler_params=pltpu.CompilerParams(use_tc_tiling_on_sc=False),
    )


def _tc_body(h0_ref, w1_ref, c1_ref, w2_ref, c2_ref, wf_ref, cf_ref, out_ref):
    h0 = h0_ref[...]
    h1 = jnp.dot(w1_ref[...], h0, preferred_element_type=jnp.float32,
                 precision=lax.Precision.HIGHEST)
    h1 = jnp.maximum(h1 + c1_ref[...], 0.0)
    h2 = jnp.dot(w2_ref[...], h1, preferred_element_type=jnp.float32,
                 precision=lax.Precision.HIGHEST)
    h2 = jnp.maximum(h2 + c2_ref[...], 0.0)
    gmf = h0[:_EMBED_DIM] * h0[_EMBED_DIM:]
    cat = jnp.concatenate([gmf, h2], axis=0)
    out = jnp.dot(wf_ref[...], cat, preferred_element_type=jnp.float32,
                  precision=lax.Precision.HIGHEST)
    out_ref[...] = out + cf_ref[...]


def _tc_mlp(h0t, w1t, c1t, w2t, c2t, wft, cft, block_b=2048):
    nblk = _BATCH // block_b
    rep2 = lambda i: (0, 0)
    return pl.pallas_call(
        _tc_body,
        grid=(nblk,),
        in_specs=[
            pl.BlockSpec((_NFEAT, block_b), lambda i: (0, i)),
            pl.BlockSpec(w1t.shape, rep2),
            pl.BlockSpec(c1t.shape, rep2),
            pl.BlockSpec(w2t.shape, rep2),
            pl.BlockSpec(c2t.shape, rep2),
            pl.BlockSpec(wft.shape, rep2),
            pl.BlockSpec(cft.shape, rep2),
        ],
        out_specs=pl.BlockSpec((1, block_b), lambda i: (0, i)),
        out_shape=jax.ShapeDtypeStruct((1, _BATCH), jnp.float32),
    )(h0t, w1t, c1t, w2t, c2t, wft, cft)


def kernel(x, table, W1, b1, g1, be1, W2, b2, g2, be2, Wf, bf):
    s = jax.lax.rsqrt(jnp.float32(1.0 + _BN_EPS))
    w1t = (W1 * (s * g1)[None, :]).T
    c1t = (b1 * s * g1 + be1)[:, None]
    w2t = (W2 * (s * g2)[None, :]).T
    c2t = (b2 * s * g2 + be2)[:, None]
    wft = Wf.T
    cft = bf[:, None]

    flat = table.T.reshape(8, 8, 15625, 128).transpose(0, 2, 1, 3).reshape(_FLAT)

    p = ((x >> 7) << 10) + (x & 127)
    pr = p.T.reshape(2, _NCHUNK, _CHUNK)

    h0t = _sc_gather()(flat, pr)
    out = _tc_mlp(h0t, w1t, c1t, w2t, c2t, wft, cft)
    return out.T
